# baseline (speedup 1.0000x reference)
import functools

import jax
import jax.numpy as jnp
import numpy as np
from jax import lax
from jax.experimental import pallas as pl
from jax.experimental.pallas import tpu as pltpu
from jax.experimental.pallas import tpu_sc as plsc

B = 32
N = 1024
KNN = 5
NC, NS, L = 2, 16, 16
NW = NC * NS
RPT = N // NW
NG = RPT // L
SUB, LANE = 8, 128
BIG = 3.0e38
SELFD = 1.0e9


def _knn_body(xs_hbm, ys_hbm, sq_hbm, out_hbm, xv, yv, sv, ov):
    wid = lax.axis_index("s") * NC + lax.axis_index("c")
    pltpu.sync_copy(xs_hbm, xv)
    pltpu.sync_copy(ys_hbm, yv)
    pltpu.sync_copy(sq_hbm, sv)

    @pl.loop(0, B)
    def _rb(b):
        @pl.loop(0, N // L)
        def _rc(c):
            sl = (b, pl.ds(c * L, L))
            for vref in (xv, yv):
                u = lax.bitcast_convert_type(vref[sl], jnp.uint32)
                u = (u + (np.uint32(0x7FFF) + ((u >> np.uint32(16))
                                               & np.uint32(1)))) & np.uint32(0xFFFF0000)
                vref[sl] = lax.bitcast_convert_type(u, jnp.float32)

    r0 = wid * RPT
    iot = lax.iota(jnp.int32, L)

    @pl.loop(0, B)
    def _batch(b):
        xq, yq, sq, rid = [], [], [], []
        for g in range(NG):
            base = r0 + g * L
            xq.append(xv[b, pl.ds(base, L)] * -2.0)
            yq.append(yv[b, pl.ds(base, L)] * -2.0)
            sq.append(sv[b, pl.ds(base, L)])
            rid.append(iot + base)

        big = jnp.full((L,), BIG, jnp.float32)
        zero = jnp.zeros((L,), jnp.int32)
        init = tuple([big] * (KNN * NG) + [zero] * (KNN * NG))

        def cbody(c, st):
            keys = [list(st[g * KNN:(g + 1) * KNN]) for g in range(NG)]
            ids = [list(st[(NG + g) * KNN:(NG + g + 1) * KNN]) for g in range(NG)]
            cb = c * L
            xc = xv[b, pl.ds(cb, L)]
            yc = yv[b, pl.ds(cb, L)]
            sc = sv[b, pl.ds(cb, L)]
            for l in range(L):
                xj = xc[l]
                yj = yc[l]
                sj = sc[l]
                jv = cb + l
                for g in range(NG):
                    d2 = (sq[g] + sj) + (xq[g] * xj + yq[g] * yj)
                    d2 = jnp.where(rid[g] == jv, SELFD, d2)
                    m = [d2 < keys[g][lvl] for lvl in range(KNN)]
                    for lvl in range(KNN - 1, 0, -1):
                        keys[g][lvl] = jnp.minimum(
                            keys[g][lvl], jnp.maximum(keys[g][lvl - 1], d2))
                        ids[g][lvl] = jnp.where(
                            m[lvl],
                            jnp.where(m[lvl - 1], ids[g][lvl - 1], jv),
                            ids[g][lvl])
                    keys[g][0] = jnp.minimum(keys[g][0], d2)
                    ids[g][0] = jnp.where(m[0], jv, ids[g][0])
            return tuple(keys[0] + keys[1] + ids[0] + ids[1])

        st = lax.fori_loop(0, N // L, cbody, init, unroll=3)
        for g in range(NG):
            for lvl in range(KNN):
                ov[b, lvl, pl.ds(g * L, L)] = st[(NG + g) * KNN + lvl]

    pltpu.sync_copy(ov, out_hbm.at[wid])


_knn_call = functools.partial(
    pl.kernel,
    out_type=jax.ShapeDtypeStruct((NW, B, KNN, RPT), jnp.int32),
    mesh=plsc.VectorSubcoreMesh(core_axis_name="c", subcore_axis_name="s"),
    scratch_types=[
        pltpu.VMEM((B, N), jnp.float32),
        pltpu.VMEM((B, N), jnp.float32),
        pltpu.VMEM((B, N), jnp.float32),
        pltpu.VMEM((B, KNN, RPT), jnp.int32),
    ],
)(_knn_body)


def _r16(v):
    u = lax.bitcast_convert_type(v, jnp.int32)
    u = (u + (np.int32(0x7FFF) + ((u >> 16) & np.int32(1)))) & np.int32(-65536)
    return lax.bitcast_convert_type(u, jnp.float32)


def _enc_body(xs_ref, ys_ref, dem_ref, cap_ref, cur_ref, dep_ref,
              wamp_ref, bamp_ref, w1_ref, b1_ref, w2_ref, b2_ref,
              feat_ref, psi_ref):
    x = xs_ref[0]
    y = ys_ref[0]
    dem = dem_ref[0]
    cap = cap_ref[0]
    pid = pl.program_id(0)
    cx, cy = cur_ref[pid, 0], cur_ref[pid, 1]
    dx, dy = dep_ref[pid, 0], dep_ref[pid, 1]
    txd, tyd = x - dx, y - dy
    d_dep = jnp.sqrt(txd * txd + tyd * tyd + 1e-12)
    txc, tyc = x - cx, y - cy
    d_cur = jnp.sqrt(txc * txc + tyc * tyc + 1e-12)
    nid = (lax.broadcasted_iota(jnp.int32, (SUB, LANE), 0) * LANE
           + lax.broadcasted_iota(jnp.int32, (SUB, LANE), 1))
    is_dep = jnp.where(nid == 0, 1.0, 0.0).astype(jnp.float32)
    dn = dem / cap
    feats = [x, y, dn, d_dep, is_dep, d_cur]
    for f in range(6):
        feat_ref[0, f] = feats[f]
    fb = [_r16(f) for f in feats]
    pa = fb[0] * wamp_ref[0, 0]
    pb = fb[0] * wamp_ref[0, 1]
    for f in range(1, 6):
        pa = pa + fb[f] * wamp_ref[f, 0]
        pb = pb + fb[f] * wamp_ref[f, 1]
    pa = pa + bamp_ref[0]
    pb = pb + bamp_ref[1]
    nrm = jnp.sqrt(pa * pa + pb * pb) + 1e-8
    pa = pa / nrm
    pb = pb / nrm
    theta = jnp.full_like(x, 0.0) + b2_ref[0]
    for j in range(16):
        h = fb[0] * w1_ref[0, j]
        for f in range(1, 6):
            h = h + fb[f] * w1_ref[f, j]
        h = h + b1_ref[j]
        theta = theta + _r16(jnp.tanh(h)) * w2_ref[j, 0]
    c, s = jnp.cos(theta), jnp.sin(theta)
    psi_ref[0, 0] = c * pa - s * pb
    psi_ref[0, 1] = s * pa + c * pb


_enc_call = pl.pallas_call(
    _enc_body,
    grid=(B,),
    in_specs=[
        pl.BlockSpec((1, SUB, LANE), lambda b: (b, 0, 0)),
        pl.BlockSpec((1, SUB, LANE), lambda b: (b, 0, 0)),
        pl.BlockSpec((1, SUB, LANE), lambda b: (b, 0, 0)),
        pl.BlockSpec(memory_space=pltpu.SMEM),
        pl.BlockSpec(memory_space=pltpu.SMEM),
        pl.BlockSpec(memory_space=pltpu.SMEM),
        pl.BlockSpec(memory_space=pltpu.SMEM),
        pl.BlockSpec(memory_space=pltpu.SMEM),
        pl.BlockSpec(memory_space=pltpu.SMEM),
        pl.BlockSpec(memory_space=pltpu.SMEM),
        pl.BlockSpec(memory_space=pltpu.SMEM),
        pl.BlockSpec(memory_space=pltpu.SMEM),
    ],
    out_specs=[
        pl.BlockSpec((1, 6, SUB, LANE), lambda b: (b, 0, 0, 0)),
        pl.BlockSpec((1, 2, SUB, LANE), lambda b: (b, 0, 0, 0)),
    ],
    out_shape=[
        jax.ShapeDtypeStruct((B, 6, SUB, LANE), jnp.float32),
        jax.ShapeDtypeStruct((B, 2, SUB, LANE), jnp.float32),
    ],
)


def kernel(coords, demands, capacity, current_node_coords,
           W_amp, b_amp, W1, b1, W2, b2):
    xs = coords[..., 0]
    ys = coords[..., 1]
    sq = jnp.sum(coords ** 2, axis=-1)
    depot = coords[:, 0, :]

    knn_t = _knn_call(xs, ys, sq)
    knn = jnp.transpose(knn_t, (1, 0, 3, 2)).reshape(B, N, KNN)

    feat_t, psi_t = _enc_call(
        xs.reshape(B, SUB, LANE), ys.reshape(B, SUB, LANE),
        demands.reshape(B, SUB, LANE), capacity, current_node_coords, depot,
        W_amp, b_amp, W1, b1, W2, b2)
    features = jnp.transpose(feat_t.reshape(B, 6, N), (0, 2, 1))
    psi_prime = jnp.transpose(psi_t.reshape(B, 2, N), (0, 2, 1))
    return psi_prime, features, knn


# speedup vs baseline: 1.0435x; 1.0435x over previous
import functools

import jax
import jax.numpy as jnp
import numpy as np
from jax import lax
from jax.experimental import pallas as pl
from jax.experimental.pallas import tpu as pltpu
from jax.experimental.pallas import tpu_sc as plsc

B = 32
N = 1024
KNN = 5
NC, NS, L = 2, 16, 16
NW = NC * NS
RPT = N // NW
NG = RPT // L
SUB, LANE = 8, 128
BIG = 3.0e38
SELFD = 1.0e9


def _knn_body(xs_hbm, ys_hbm, sq_hbm, out_hbm, xv, yv, sv, ov):
    wid = lax.axis_index("s") * NC + lax.axis_index("c")
    pltpu.sync_copy(xs_hbm, xv)
    pltpu.sync_copy(ys_hbm, yv)
    pltpu.sync_copy(sq_hbm, sv)

    @pl.loop(0, B)
    def _rb(b):
        @pl.loop(0, N // L)
        def _rc(c):
            sl = (b, pl.ds(c * L, L))
            for vref in (xv, yv):
                u = lax.bitcast_convert_type(vref[sl], jnp.uint32)
                u = (u + (np.uint32(0x7FFF) + ((u >> np.uint32(16))
                                               & np.uint32(1)))) & np.uint32(0xFFFF0000)
                vref[sl] = lax.bitcast_convert_type(u, jnp.float32)

    r0 = wid * RPT
    iot = lax.iota(jnp.int32, L)

    @pl.loop(0, B)
    def _batch(b):
        xq, yq, sq, rid = [], [], [], []
        for g in range(NG):
            base = r0 + g * L
            xq.append(xv[b, pl.ds(base, L)] * -2.0)
            yq.append(yv[b, pl.ds(base, L)] * -2.0)
            sq.append(sv[b, pl.ds(base, L)])
            rid.append(iot + base)

        big = jnp.full((L,), BIG, jnp.float32)
        zero = jnp.zeros((L,), jnp.int32)
        init = tuple([big] * (KNN * NG) + [zero] * (KNN * NG))

        def cbody(c, st):
            keys = [list(st[g * KNN:(g + 1) * KNN]) for g in range(NG)]
            ids = [list(st[(NG + g) * KNN:(NG + g + 1) * KNN]) for g in range(NG)]
            cb = c * L
            xc = xv[b, pl.ds(cb, L)]
            yc = yv[b, pl.ds(cb, L)]
            sc = sv[b, pl.ds(cb, L)]
            for l in range(L):
                xj = xc[l]
                yj = yc[l]
                sj = sc[l]
                jv = cb + l
                for g in range(NG):
                    d2 = (sq[g] + sj) + (xq[g] * xj + yq[g] * yj)
                    d2 = jnp.where(rid[g] == jv, SELFD, d2)
                    m = [d2 < keys[g][lvl] for lvl in range(KNN)]
                    for lvl in range(KNN - 1, 0, -1):
                        keys[g][lvl] = jnp.minimum(
                            keys[g][lvl], jnp.maximum(keys[g][lvl - 1], d2))
                        ids[g][lvl] = jnp.where(
                            m[lvl],
                            jnp.where(m[lvl - 1], ids[g][lvl - 1], jv),
                            ids[g][lvl])
                    keys[g][0] = jnp.minimum(keys[g][0], d2)
                    ids[g][0] = jnp.where(m[0], jv, ids[g][0])
            return tuple(keys[0] + keys[1] + ids[0] + ids[1])

        st = lax.fori_loop(0, N // L, cbody, init, unroll=2)
        for g in range(NG):
            for lvl in range(KNN):
                ov[b, lvl, pl.ds(g * L, L)] = st[(NG + g) * KNN + lvl]

    pltpu.sync_copy(ov, out_hbm.at[wid])


_knn_call = functools.partial(
    pl.kernel,
    out_type=jax.ShapeDtypeStruct((NW, B, KNN, RPT), jnp.int32),
    mesh=plsc.VectorSubcoreMesh(core_axis_name="c", subcore_axis_name="s"),
    scratch_types=[
        pltpu.VMEM((B, N), jnp.float32),
        pltpu.VMEM((B, N), jnp.float32),
        pltpu.VMEM((B, N), jnp.float32),
        pltpu.VMEM((B, KNN, RPT), jnp.int32),
    ],
)(_knn_body)


def _r16(v):
    u = lax.bitcast_convert_type(v, jnp.int32)
    u = (u + (np.int32(0x7FFF) + ((u >> 16) & np.int32(1)))) & np.int32(-65536)
    return lax.bitcast_convert_type(u, jnp.float32)


def _enc_body(xs_ref, ys_ref, dem_ref, cap_ref, cur_ref, dep_ref,
              wamp_ref, bamp_ref, w1_ref, b1_ref, w2_ref, b2_ref,
              feat_ref, psi_ref):
    x = xs_ref[0]
    y = ys_ref[0]
    dem = dem_ref[0]
    cap = cap_ref[0]
    pid = pl.program_id(0)
    cx, cy = cur_ref[pid, 0], cur_ref[pid, 1]
    dx, dy = dep_ref[pid, 0], dep_ref[pid, 1]
    txd, tyd = x - dx, y - dy
    d_dep = jnp.sqrt(txd * txd + tyd * tyd + 1e-12)
    txc, tyc = x - cx, y - cy
    d_cur = jnp.sqrt(txc * txc + tyc * tyc + 1e-12)
    nid = (lax.broadcasted_iota(jnp.int32, (SUB, LANE), 0) * LANE
           + lax.broadcasted_iota(jnp.int32, (SUB, LANE), 1))
    is_dep = jnp.where(nid == 0, 1.0, 0.0).astype(jnp.float32)
    dn = dem / cap
    feats = [x, y, dn, d_dep, is_dep, d_cur]
    for f in range(6):
        feat_ref[0, f] = feats[f]
    fb = [_r16(f) for f in feats]
    pa = fb[0] * wamp_ref[0, 0]
    pb = fb[0] * wamp_ref[0, 1]
    for f in range(1, 6):
        pa = pa + fb[f] * wamp_ref[f, 0]
        pb = pb + fb[f] * wamp_ref[f, 1]
    pa = pa + bamp_ref[0]
    pb = pb + bamp_ref[1]
    nrm = jnp.sqrt(pa * pa + pb * pb) + 1e-8
    pa = pa / nrm
    pb = pb / nrm
    theta = jnp.full_like(x, 0.0) + b2_ref[0]
    for j in range(16):
        h = fb[0] * w1_ref[0, j]
        for f in range(1, 6):
            h = h + fb[f] * w1_ref[f, j]
        h = h + b1_ref[j]
        theta = theta + _r16(jnp.tanh(h)) * w2_ref[j, 0]
    c, s = jnp.cos(theta), jnp.sin(theta)
    psi_ref[0, 0] = c * pa - s * pb
    psi_ref[0, 1] = s * pa + c * pb


_enc_call = pl.pallas_call(
    _enc_body,
    grid=(B,),
    in_specs=[
        pl.BlockSpec((1, SUB, LANE), lambda b: (b, 0, 0)),
        pl.BlockSpec((1, SUB, LANE), lambda b: (b, 0, 0)),
        pl.BlockSpec((1, SUB, LANE), lambda b: (b, 0, 0)),
        pl.BlockSpec(memory_space=pltpu.SMEM),
        pl.BlockSpec(memory_space=pltpu.SMEM),
        pl.BlockSpec(memory_space=pltpu.SMEM),
        pl.BlockSpec(memory_space=pltpu.SMEM),
        pl.BlockSpec(memory_space=pltpu.SMEM),
        pl.BlockSpec(memory_space=pltpu.SMEM),
        pl.BlockSpec(memory_space=pltpu.SMEM),
        pl.BlockSpec(memory_space=pltpu.SMEM),
        pl.BlockSpec(memory_space=pltpu.SMEM),
    ],
    out_specs=[
        pl.BlockSpec((1, 6, SUB, LANE), lambda b: (b, 0, 0, 0)),
        pl.BlockSpec((1, 2, SUB, LANE), lambda b: (b, 0, 0, 0)),
    ],
    out_shape=[
        jax.ShapeDtypeStruct((B, 6, SUB, LANE), jnp.float32),
        jax.ShapeDtypeStruct((B, 2, SUB, LANE), jnp.float32),
    ],
)


def kernel(coords, demands, capacity, current_node_coords,
           W_amp, b_amp, W1, b1, W2, b2):
    xs = coords[..., 0]
    ys = coords[..., 1]
    sq = jnp.sum(coords ** 2, axis=-1)
    depot = coords[:, 0, :]

    knn_t = _knn_call(xs, ys, sq)
    knn = jnp.transpose(knn_t, (1, 0, 3, 2)).reshape(B, N, KNN)

    feat_t, psi_t = _enc_call(
        xs.reshape(B, SUB, LANE), ys.reshape(B, SUB, LANE),
        demands.reshape(B, SUB, LANE), capacity, current_node_coords, depot,
        W_amp, b_amp, W1, b1, W2, b2)
    features = jnp.transpose(feat_t.reshape(B, 6, N), (0, 2, 1))
    psi_prime = jnp.transpose(psi_t.reshape(B, 2, N), (0, 2, 1))
    return psi_prime, features, knn
